# baseline (speedup 1.0000x reference)
import functools

import jax
import jax.numpy as jnp
from jax import lax
from jax.experimental import pallas as pl
from jax.experimental.pallas import tpu as pltpu
from jax.experimental.pallas import tpu_sc as plsc

_N_NODES = 10000
_NE = 160000
_D = 256
_HID = 64
_L = 4
_NPAD = 10240
_NTILES = 32
_NPT = _NPAD // _NTILES
_CHUNK = 1600
_NCHUNKS = _NE // _CHUNK
_GB = 64
_BM = 512


_LCAP = _NE + _GB
_BUF = _GB + _CHUNK + 16
_SENT = _NPT


def _edge_lists(src, dst):
    mesh = plsc.VectorSubcoreMesh(core_axis_name="c", subcore_axis_name="s")

    @functools.partial(
        pl.kernel,
        out_type=[
            jax.ShapeDtypeStruct((_NTILES, _LCAP), jnp.int32),
            jax.ShapeDtypeStruct((_NTILES, _LCAP), jnp.int32),
            jax.ShapeDtypeStruct((_NTILES, 16), jnp.int32),
        ],
        mesh=mesh,
        scratch_types=[
            pltpu.VMEM((_CHUNK,), jnp.int32),
            pltpu.VMEM((_CHUNK,), jnp.int32),
            pltpu.VMEM((_BUF,), jnp.int32),
            pltpu.VMEM((_BUF,), jnp.int32),
            pltpu.VMEM((16,), jnp.int32),
        ],
        compiler_params=pltpu.CompilerParams(needs_layout_passes=False),
    )
    def list_kernel(src_hbm, dst_hbm, csrc_hbm, cdst_hbm, cnt_hbm,
                    srcv, dstv, bsrc, bdst, cbuf):
        wid = lax.axis_index("s") * 2 + lax.axis_index("c")
        lo = wid * _NPT

        def chunk_body(c, st):
            wr, cur = st
            pltpu.sync_copy(src_hbm.at[pl.ds(c * _CHUNK, _CHUNK)], srcv)
            pltpu.sync_copy(dst_hbm.at[pl.ds(c * _CHUNK, _CHUNK)], dstv)

            def grp(g, cur2):
                dv = dstv[pl.ds(g * 16, 16)]
                sv = srcv[pl.ds(g * 16, 16)]
                m = (dv >= lo) & (dv < lo + _NPT)
                pfx = plsc.cumsum(jnp.where(m, 1, 0))
                pos = cur2 + pfx - 1
                plsc.store_scatter(bsrc, [pos], sv, mask=m)
                plsc.store_scatter(bdst, [pos], dv - lo, mask=m)
                return cur2 + pfx[15]

            cur = lax.fori_loop(0, _CHUNK // 16, grp, cur)

            def flush_cond(st2):
                return st2[1] >= _GB

            def flush(st2):
                wr2, cur2 = st2
                wra = pl.multiple_of(wr2, _GB)
                pltpu.sync_copy(bsrc.at[pl.ds(0, _GB)],
                                csrc_hbm.at[wid, pl.ds(wra, _GB)])
                pltpu.sync_copy(bdst.at[pl.ds(0, _GB)],
                                cdst_hbm.at[wid, pl.ds(wra, _GB)])
                ngrp = (cur2 - _GB + 15) // 16

                def shift(t, carry):
                    v = bsrc[pl.ds(_GB + t * 16, 16)]
                    bsrc[pl.ds(t * 16, 16)] = v
                    w = bdst[pl.ds(_GB + t * 16, 16)]
                    bdst[pl.ds(t * 16, 16)] = w
                    return carry

                lax.fori_loop(0, ngrp, shift, 0)
                return (wr2 + _GB, cur2 - _GB)

            wr, cur = lax.while_loop(flush_cond, flush, (wr, cur))
            return (wr, cur)

        wr, cur = lax.fori_loop(0, _NCHUNKS, chunk_body, (0, 0))
        zi = jnp.zeros((16,), jnp.int32)
        sent = jnp.full((16,), _SENT, jnp.int32)
        lanes = jnp.arange(16, dtype=jnp.int32)
        for t in range(_GB // 16):
            plsc.store_scatter(bsrc, [cur + t * 16 + lanes], zi)
            plsc.store_scatter(bdst, [cur + t * 16 + lanes], sent)
        wra = pl.multiple_of(wr, _GB)
        pltpu.sync_copy(bsrc.at[pl.ds(0, _GB)],
                        csrc_hbm.at[wid, pl.ds(wra, _GB)])
        pltpu.sync_copy(bdst.at[pl.ds(0, _GB)],
                        cdst_hbm.at[wid, pl.ds(wra, _GB)])
        nb = wr // _GB + 1
        cbuf[pl.ds(0, 16)] = jnp.zeros((16,), jnp.int32) + nb
        pltpu.sync_copy(cbuf, cnt_hbm.at[wid])

    return list_kernel(src, dst)


_DH = _D // 2


def _segmax(h, csrc, cdst, cnt):
    mesh = plsc.VectorSubcoreMesh(core_axis_name="c", subcore_axis_name="s")
    nacch = (_NPT + 1) * _DH

    @functools.partial(
        pl.kernel,
        out_type=[
            jax.ShapeDtypeStruct((_NPAD, _DH), jnp.float32),
            jax.ShapeDtypeStruct((_NPAD, _DH), jnp.float32),
        ],
        mesh=mesh,
        scratch_types=[
            pltpu.VMEM((_NPT + 1, _DH), jnp.float32),
            pltpu.VMEM((_NPT + 1, _DH), jnp.float32),
            pltpu.VMEM((_GB,), jnp.int32),
            pltpu.VMEM((_GB,), jnp.int32),
            pltpu.VMEM((_GB,), jnp.int32),
            pltpu.VMEM((_GB,), jnp.int32),
            pltpu.VMEM((_GB, _D), jnp.float32),
            pltpu.VMEM((_GB, _D), jnp.float32),
            pltpu.VMEM((16,), jnp.int32),
            pltpu.SemaphoreType.DMA,
            pltpu.SemaphoreType.DMA,
            pltpu.SemaphoreType.DMA,
            pltpu.SemaphoreType.DMA,
            pltpu.SemaphoreType.DMA,
            pltpu.SemaphoreType.DMA,
        ],
        compiler_params=pltpu.CompilerParams(needs_layout_passes=False),
    )
    def seg_kernel(h_hbm, csrc_hbm, cdst_hbm, cnt_hbm, outa_hbm, outb_hbm,
                   acca, accb, idx0, idx1, db0, db1, rows0, rows1, cbuf,
                   semr0, semr1, semia0, semia1, semib0, semib1):
        wid = lax.axis_index("s") * 2 + lax.axis_index("c")
        zf = jnp.zeros((16,), jnp.float32)

        def initrow(r, carry):
            for j in range(_DH // 16):
                acca[r, pl.ds(j * 16, 16)] = zf
                accb[r, pl.ds(j * 16, 16)] = zf
            return carry

        lax.fori_loop(0, _NPT + 1, initrow, 0)

        pltpu.sync_copy(cnt_hbm.at[wid], cbuf)
        nb = cbuf[pl.ds(0, 16)][0]

        idxs = (idx0, idx1)
        dbs = (db0, db1)
        rows = (rows0, rows1)
        semr = (semr0, semr1)
        semia = (semia0, semia1)
        semib = (semib0, semib1)
        lanes = jnp.arange(16, dtype=jnp.int32)
        cols = [lanes + j * 16 for j in range(_DH // 16)]
        nj = _DH // 16

        def stage_idx(t, s):
            pltpu.async_copy(csrc_hbm.at[wid, pl.ds(t * _GB, _GB)],
                             idxs[s], semia[s])

        def stage_dloc(t, s):
            pltpu.async_copy(cdst_hbm.at[wid, pl.ds(t * _GB, _GB)],
                             dbs[s], semib[s])

        def fire(t, s):
            pltpu.make_async_copy(csrc_hbm.at[wid, pl.ds(t * _GB, _GB)],
                                  idxs[s], semia[s]).wait()
            pltpu.async_copy(h_hbm.at[idxs[s]], rows[s], semr[s])

        def wait_rows(t, s):
            pltpu.make_async_copy(h_hbm.at[idxs[s]], rows[s], semr[s]).wait()

        def update(t, s):
            pltpu.make_async_copy(cdst_hbm.at[wid, pl.ds(t * _GB, _GB)],
                                  dbs[s], semib[s]).wait()

            def rmw(half, d, e, feat0):
                olds = [half[d, pl.ds(j * 16, 16)] for j in range(nj)]
                for j in range(nj):
                    nv = jnp.maximum(
                        olds[j], rows[s][e, pl.ds(feat0 + j * 16, 16)])
                    half[d, pl.ds(j * 16, 16)] = nv

            def sub(g, carry):
                for li in range(0, 16, 2):
                    ea = g * 16 + li
                    eb = ea + 1
                    da = dbs[s][pl.ds(ea, 16)][0]
                    db_ = dbs[s][pl.ds(eb, 16)][0]
                    rmw(acca, da, ea, 0)
                    rmw(accb, db_, eb, _DH)
                    rmw(accb, da, ea, _DH)
                    rmw(acca, db_, eb, 0)
                return carry

            lax.fori_loop(0, _GB // 16, sub, 0)

        stage_idx(0, 0)
        stage_dloc(0, 0)
        fire(0, 0)

        @pl.when(1 < nb)
        def _():
            stage_idx(1, 1)
            stage_dloc(1, 1)

        def pair(bb, carry):
            t0 = bb * 2

            @pl.when(t0 + 1 < nb)
            def _():
                fire(t0 + 1, 1)

            wait_rows(t0, 0)

            @pl.when(t0 + 2 < nb)
            def _():
                stage_idx(t0 + 2, 0)

            update(t0, 0)

            @pl.when(t0 + 2 < nb)
            def _():
                stage_dloc(t0 + 2, 0)

            @pl.when(t0 + 1 < nb)
            def _():
                @pl.when(t0 + 2 < nb)
                def _():
                    fire(t0 + 2, 0)

                wait_rows(t0 + 1, 1)

                @pl.when(t0 + 3 < nb)
                def _():
                    stage_idx(t0 + 3, 1)

                update(t0 + 1, 1)

                @pl.when(t0 + 3 < nb)
                def _():
                    stage_dloc(t0 + 3, 1)

            return carry

        lax.fori_loop(0, (nb + 1) // 2, pair, 0)
        pltpu.sync_copy(acca.at[pl.ds(0, _NPT)],
                        outa_hbm.at[pl.ds(wid * _NPT, _NPT)])
        pltpu.sync_copy(accb.at[pl.ds(0, _NPT)],
                        outb_hbm.at[pl.ds(wid * _NPT, _NPT)])

    return seg_kernel(h, csrc, cdst, cnt)


def _mm_relu(x, W, b):
    def body(x_ref, w_ref, b_ref, o_ref):
        o_ref[...] = jnp.maximum(
            jnp.dot(x_ref[...], w_ref[...],
                    preferred_element_type=jnp.float32) + b_ref[...], 0.0)

    n = x.shape[0]
    return pl.pallas_call(
        body,
        grid=(n // _BM,),
        in_specs=[
            pl.BlockSpec((_BM, _D), lambda i: (i, 0)),
            pl.BlockSpec((_D, _D), lambda i: (0, 0)),
            pl.BlockSpec((1, _D), lambda i: (0, 0)),
        ],
        out_specs=pl.BlockSpec((_BM, _D), lambda i: (i, 0)),
        out_shape=jax.ShapeDtypeStruct((n, _D), jnp.float32),
    )(x, W, b.reshape(1, _D))


def _mm_out(x, pa, pb, Ws, WnA, WnB, bo):
    def body(x_ref, pa_ref, pb_ref, ws_ref, wna_ref, wnb_ref, b_ref, o_ref):
        acc = jnp.dot(x_ref[...], ws_ref[...],
                      preferred_element_type=jnp.float32)
        acc = acc + jnp.dot(pa_ref[...], wna_ref[...],
                            preferred_element_type=jnp.float32)
        acc = acc + jnp.dot(pb_ref[...], wnb_ref[...],
                            preferred_element_type=jnp.float32)
        o_ref[...] = jnp.maximum(acc + b_ref[...], 0.0)

    n = x.shape[0]
    return pl.pallas_call(
        body,
        grid=(n // _BM,),
        in_specs=[
            pl.BlockSpec((_BM, _D), lambda i: (i, 0)),
            pl.BlockSpec((_BM, _DH), lambda i: (i, 0)),
            pl.BlockSpec((_BM, _DH), lambda i: (i, 0)),
            pl.BlockSpec((_D, _D), lambda i: (0, 0)),
            pl.BlockSpec((_DH, _D), lambda i: (0, 0)),
            pl.BlockSpec((_DH, _D), lambda i: (0, 0)),
            pl.BlockSpec((1, _D), lambda i: (0, 0)),
        ],
        out_specs=pl.BlockSpec((_BM, _D), lambda i: (i, 0)),
        out_shape=jax.ShapeDtypeStruct((n, _D), jnp.float32),
    )(x, pa, pb, Ws, WnA, WnB, bo.reshape(1, _D))


def _loss_call(SA, SB, gt, W1, b1c, W2, b2c):
    Nr, Mr = gt.shape

    def body(sa_ref, sb_ref, gt_ref, w1_ref, b1_ref, w2_ref, b2_ref,
             pred_ref, aff_ref, trip_ref, bm_ref):
        SA_ = sa_ref[...]
        SB_ = sb_ref[...]
        gt_ = gt_ref[...]
        w1 = w1_ref[...]
        A1T = lax.dot_general(w1, SA_, (((0,), (1,)), ((), ())),
                              preferred_element_type=jnp.float32,
                              precision=lax.Precision.HIGHEST)
        A1T = A1T + b1_ref[...]
        bm_ref[...] = jnp.dot(SB_, w1, preferred_element_type=jnp.float32,
                              precision=lax.Precision.HIGHEST)
        w2c = w2_ref[...]
        b2s = b2_ref[0, 0]

        def iblk(ib, aff_sum):
            i0 = ib * 8
            Bblk = bm_ref[pl.ds(i0, 8), :][:, :, None]
            Z = A1T[None, :, :] - Bblk
            T = jnp.maximum(Z, 0.0) * w2c[None, :, :]
            P = jnp.sum(T, axis=1) + b2s
            pred_ref[pl.ds(i0, 8), :] = P
            g = gt_ref[pl.ds(i0, 8), :]
            return aff_sum + jnp.sum(
                jnp.maximum(P, 0.0) - P * g
                + jnp.log1p(jnp.exp(-jnp.abs(P))))

        aff_sum = lax.fori_loop(0, Nr // 8, iblk, jnp.float32(0.0))
        aff_ref[0, 0] = aff_sum / jnp.float32(Nr * Mr)

        ra = jnp.sum(SB_ * SB_, axis=1)[:, None]
        rb = jnp.sum(SA_ * SA_, axis=1)[None, :]
        G = lax.dot_general(SB_, SA_, (((1,), (1,)), ((), ())),
                            preferred_element_type=jnp.float32,
                            precision=lax.Precision.HIGHEST)
        d2 = ra + rb - 2.0 * G
        Dm = jnp.sqrt(jnp.maximum(d2, 1e-12))
        d_pos = jnp.max(Dm * gt_, axis=1)
        big = jnp.float32(3.0e38)
        d_neg = jnp.min(jnp.where(gt_ < 0.5, Dm, big), axis=1)
        d_neg = jnp.where(d_neg > jnp.float32(1.0e37), 0.0, d_neg)
        trip_ref[0, 0] = jnp.mean(jnp.maximum(d_pos - d_neg + 10.0, 0.0))

    return pl.pallas_call(
        body,
        in_specs=[
            pl.BlockSpec((Nr, _D), lambda: (0, 0)),
            pl.BlockSpec((Nr, _D), lambda: (0, 0)),
            pl.BlockSpec((Nr, Mr), lambda: (0, 0)),
            pl.BlockSpec((_D, _HID), lambda: (0, 0)),
            pl.BlockSpec((_HID, 1), lambda: (0, 0)),
            pl.BlockSpec((_HID, 1), lambda: (0, 0)),
            pl.BlockSpec((1, 1), lambda: (0, 0)),
        ],
        out_specs=[
            pl.BlockSpec((Nr, Mr), lambda: (0, 0)),
            pl.BlockSpec(memory_space=pltpu.SMEM),
            pl.BlockSpec(memory_space=pltpu.SMEM),
        ],
        out_shape=[
            jax.ShapeDtypeStruct((Nr, Mr), jnp.float32),
            jax.ShapeDtypeStruct((1, 1), jnp.float32),
            jax.ShapeDtypeStruct((1, 1), jnp.float32),
        ],
        scratch_shapes=[pltpu.VMEM((Nr, _HID), jnp.float32)],
    )(SA, SB, gt, W1, b1c, W2, b2c)


def kernel(embeddings, gt_aff_mat, edge_index, N, M, W_pool, b_pool, W_self,
           W_neigh, b_out, W1, b1, W2, b2):
    src = edge_index[0]
    dst = edge_index[1]
    Nr, Mr = gt_aff_mat.shape
    xp = jnp.zeros((_NPAD, _D), jnp.float32).at[:_N_NODES].set(embeddings)
    b1c = b1.reshape(_HID, 1)
    b2c = b2.reshape(1, 1)
    aff = jnp.float32(0.0)
    trip = jnp.float32(0.0)
    pred = None
    csrc, cdst, cnt = _edge_lists(src, dst)
    for l in range(_L):
        h = _mm_relu(xp, W_pool[l], b_pool[l])
        pa, pb = _segmax(h, csrc, cdst, cnt)
        xp = _mm_out(xp, pa, pb, W_self[l], W_neigh[l][:_DH],
                     W_neigh[l][_DH:], b_out[l])
        SA = lax.dynamic_slice(xp, (N, 0), (Mr, _D))
        SB = lax.dynamic_slice(xp, (M - Mr, 0), (Nr, _D))
        pred, aff_l, trip_l = _loss_call(SA, SB, gt_aff_mat, W1, b1c, W2, b2c)
        aff = aff + aff_l[0, 0]
        trip = trip + trip_l[0, 0]
    total = trip + aff
    return (total, trip, aff, pred)


# speedup vs baseline: 1.0087x; 1.0087x over previous
import functools

import jax
import jax.numpy as jnp
from jax import lax
from jax.experimental import pallas as pl
from jax.experimental.pallas import tpu as pltpu
from jax.experimental.pallas import tpu_sc as plsc

_N_NODES = 10000
_NE = 160000
_D = 256
_HID = 64
_L = 4
_NPAD = 10240
_NTILES = 32
_NPT = _NPAD // _NTILES
_CHUNK = 1600
_NCHUNKS = _NE // _CHUNK
_GB = 64
_BM = 512


_LCAP = _NE + _GB
_BUF = _GB + _CHUNK + 16
_SENT = _NPT


def _edge_lists(src, dst):
    mesh = plsc.VectorSubcoreMesh(core_axis_name="c", subcore_axis_name="s")

    @functools.partial(
        pl.kernel,
        out_type=[
            jax.ShapeDtypeStruct((_NTILES, _LCAP), jnp.int32),
            jax.ShapeDtypeStruct((_NTILES, _LCAP), jnp.int32),
            jax.ShapeDtypeStruct((_NTILES, 16), jnp.int32),
        ],
        mesh=mesh,
        scratch_types=[
            pltpu.VMEM((_CHUNK,), jnp.int32),
            pltpu.VMEM((_CHUNK,), jnp.int32),
            pltpu.VMEM((_BUF,), jnp.int32),
            pltpu.VMEM((_BUF,), jnp.int32),
            pltpu.VMEM((16,), jnp.int32),
        ],
        compiler_params=pltpu.CompilerParams(needs_layout_passes=False),
    )
    def list_kernel(src_hbm, dst_hbm, csrc_hbm, cdst_hbm, cnt_hbm,
                    srcv, dstv, bsrc, bdst, cbuf):
        wid = lax.axis_index("s") * 2 + lax.axis_index("c")
        lo = wid * _NPT

        def chunk_body(c, st):
            wr, cur = st
            pltpu.sync_copy(src_hbm.at[pl.ds(c * _CHUNK, _CHUNK)], srcv)
            pltpu.sync_copy(dst_hbm.at[pl.ds(c * _CHUNK, _CHUNK)], dstv)

            def grp(g, cur2):
                dv = dstv[pl.ds(g * 16, 16)]
                sv = srcv[pl.ds(g * 16, 16)]
                m = (dv >= lo) & (dv < lo + _NPT)
                pfx = plsc.cumsum(jnp.where(m, 1, 0))
                pos = cur2 + pfx - 1
                plsc.store_scatter(bsrc, [pos], sv, mask=m)
                plsc.store_scatter(bdst, [pos], dv - lo, mask=m)
                return cur2 + pfx[15]

            cur = lax.fori_loop(0, _CHUNK // 16, grp, cur)

            def flush_cond(st2):
                return st2[1] >= _GB

            def flush(st2):
                wr2, cur2 = st2
                wra = pl.multiple_of(wr2, _GB)
                pltpu.sync_copy(bsrc.at[pl.ds(0, _GB)],
                                csrc_hbm.at[wid, pl.ds(wra, _GB)])
                pltpu.sync_copy(bdst.at[pl.ds(0, _GB)],
                                cdst_hbm.at[wid, pl.ds(wra, _GB)])
                ngrp = (cur2 - _GB + 15) // 16

                def shift(t, carry):
                    v = bsrc[pl.ds(_GB + t * 16, 16)]
                    bsrc[pl.ds(t * 16, 16)] = v
                    w = bdst[pl.ds(_GB + t * 16, 16)]
                    bdst[pl.ds(t * 16, 16)] = w
                    return carry

                lax.fori_loop(0, ngrp, shift, 0)
                return (wr2 + _GB, cur2 - _GB)

            wr, cur = lax.while_loop(flush_cond, flush, (wr, cur))
            return (wr, cur)

        wr, cur = lax.fori_loop(0, _NCHUNKS, chunk_body, (0, 0))
        zi = jnp.zeros((16,), jnp.int32)
        sent = jnp.full((16,), _SENT, jnp.int32)
        lanes = jnp.arange(16, dtype=jnp.int32)
        for t in range(_GB // 16):
            plsc.store_scatter(bsrc, [cur + t * 16 + lanes], zi)
            plsc.store_scatter(bdst, [cur + t * 16 + lanes], sent)
        wra = pl.multiple_of(wr, _GB)
        pltpu.sync_copy(bsrc.at[pl.ds(0, _GB)],
                        csrc_hbm.at[wid, pl.ds(wra, _GB)])
        pltpu.sync_copy(bdst.at[pl.ds(0, _GB)],
                        cdst_hbm.at[wid, pl.ds(wra, _GB)])
        nb = wr // _GB + 1
        cbuf[pl.ds(0, 16)] = jnp.zeros((16,), jnp.int32) + nb
        pltpu.sync_copy(cbuf, cnt_hbm.at[wid])

    return list_kernel(src, dst)


_DH = _D // 2


def _segmax(h, csrc, cdst, cnt):
    mesh = plsc.VectorSubcoreMesh(core_axis_name="c", subcore_axis_name="s")
    nacch = (_NPT + 1) * _DH

    @functools.partial(
        pl.kernel,
        out_type=[
            jax.ShapeDtypeStruct((_NPAD * _DH,), jnp.float32),
            jax.ShapeDtypeStruct((_NPAD * _DH,), jnp.float32),
        ],
        mesh=mesh,
        scratch_types=[
            pltpu.VMEM((nacch,), jnp.float32),
            pltpu.VMEM((nacch,), jnp.float32),
            pltpu.VMEM((_GB,), jnp.int32),
            pltpu.VMEM((_GB,), jnp.int32),
            pltpu.VMEM((_GB,), jnp.int32),
            pltpu.VMEM((_GB,), jnp.int32),
            pltpu.VMEM((_GB, _D), jnp.float32),
            pltpu.VMEM((_GB, _D), jnp.float32),
            pltpu.VMEM((16,), jnp.int32),
            pltpu.SemaphoreType.DMA,
            pltpu.SemaphoreType.DMA,
            pltpu.SemaphoreType.DMA,
            pltpu.SemaphoreType.DMA,
            pltpu.SemaphoreType.DMA,
            pltpu.SemaphoreType.DMA,
        ],
        compiler_params=pltpu.CompilerParams(needs_layout_passes=False),
    )
    def seg_kernel(h_hbm, csrc_hbm, cdst_hbm, cnt_hbm, outa_hbm, outb_hbm,
                   acca, accb, idx0, idx1, db0, db1, rows0, rows1, cbuf,
                   semr0, semr1, semia0, semia1, semib0, semib1):
        wid = lax.axis_index("s") * 2 + lax.axis_index("c")
        zf = jnp.zeros((16,), jnp.float32)

        def initrow(r, carry):
            for j in range(_DH // 16):
                acca[pl.ds(r * _DH + j * 16, 16)] = zf
                accb[pl.ds(r * _DH + j * 16, 16)] = zf
            return carry

        lax.fori_loop(0, _NPT + 1, initrow, 0)

        pltpu.sync_copy(cnt_hbm.at[wid], cbuf)
        nb = cbuf[pl.ds(0, 16)][0]

        idxs = (idx0, idx1)
        dbs = (db0, db1)
        rows = (rows0, rows1)
        semr = (semr0, semr1)
        semia = (semia0, semia1)
        semib = (semib0, semib1)
        lanes = jnp.arange(16, dtype=jnp.int32)
        cols = [lanes + j * 16 for j in range(_DH // 16)]
        nj = _DH // 16

        def stage_idx(t, s):
            pltpu.async_copy(csrc_hbm.at[wid, pl.ds(t * _GB, _GB)],
                             idxs[s], semia[s])

        def stage_dloc(t, s):
            pltpu.async_copy(cdst_hbm.at[wid, pl.ds(t * _GB, _GB)],
                             dbs[s], semib[s])

        def fire(t, s):
            pltpu.make_async_copy(csrc_hbm.at[wid, pl.ds(t * _GB, _GB)],
                                  idxs[s], semia[s]).wait()
            pltpu.async_copy(h_hbm.at[idxs[s]], rows[s], semr[s])

        def wait_rows(t, s):
            pltpu.make_async_copy(h_hbm.at[idxs[s]], rows[s], semr[s]).wait()

        def update(t, s):
            pltpu.make_async_copy(cdst_hbm.at[wid, pl.ds(t * _GB, _GB)],
                                  dbs[s], semib[s]).wait()

            def rmw(half, ad, e, feat0):
                olds = [plsc.load_gather(half, [ad[j]]) for j in range(nj)]
                for j in range(nj):
                    nv = jnp.maximum(
                        olds[j], rows[s][e, pl.ds(feat0 + j * 16, 16)])
                    plsc.store_scatter(half, [ad[j]], nv)

            def sub(g, carry):
                for li in range(0, 16, 2):
                    ea = g * 16 + li
                    eb = ea + 1
                    da = plsc.load_gather(
                        dbs[s], [jnp.zeros((16,), jnp.int32) + ea])
                    db_ = plsc.load_gather(
                        dbs[s], [jnp.zeros((16,), jnp.int32) + eb])
                    ada = [da * _DH + cols[j] for j in range(nj)]
                    adb = [db_ * _DH + cols[j] for j in range(nj)]
                    rmw(acca, ada, ea, 0)
                    rmw(accb, adb, eb, _DH)
                    rmw(accb, ada, ea, _DH)
                    rmw(acca, adb, eb, 0)
                return carry

            lax.fori_loop(0, _GB // 16, sub, 0)

        stage_idx(0, 0)
        stage_dloc(0, 0)
        fire(0, 0)

        @pl.when(1 < nb)
        def _():
            stage_idx(1, 1)
            stage_dloc(1, 1)

        def pair(bb, carry):
            t0 = bb * 2

            @pl.when(t0 + 1 < nb)
            def _():
                fire(t0 + 1, 1)

            wait_rows(t0, 0)

            @pl.when(t0 + 2 < nb)
            def _():
                stage_idx(t0 + 2, 0)

            update(t0, 0)

            @pl.when(t0 + 2 < nb)
            def _():
                stage_dloc(t0 + 2, 0)

            @pl.when(t0 + 1 < nb)
            def _():
                @pl.when(t0 + 2 < nb)
                def _():
                    fire(t0 + 2, 0)

                wait_rows(t0 + 1, 1)

                @pl.when(t0 + 3 < nb)
                def _():
                    stage_idx(t0 + 3, 1)

                update(t0 + 1, 1)

                @pl.when(t0 + 3 < nb)
                def _():
                    stage_dloc(t0 + 3, 1)

            return carry

        lax.fori_loop(0, (nb + 1) // 2, pair, 0)
        pltpu.sync_copy(acca.at[pl.ds(0, _NPT * _DH)],
                        outa_hbm.at[pl.ds(wid * _NPT * _DH, _NPT * _DH)])
        pltpu.sync_copy(accb.at[pl.ds(0, _NPT * _DH)],
                        outb_hbm.at[pl.ds(wid * _NPT * _DH, _NPT * _DH)])

    pa, pb = seg_kernel(h, csrc, cdst, cnt)
    return pa.reshape(_NPAD, _DH), pb.reshape(_NPAD, _DH)


def _mm_relu(x, W, b):
    def body(x_ref, w_ref, b_ref, o_ref):
        o_ref[...] = jnp.maximum(
            jnp.dot(x_ref[...], w_ref[...],
                    preferred_element_type=jnp.float32) + b_ref[...], 0.0)

    n = x.shape[0]
    return pl.pallas_call(
        body,
        grid=(n // _BM,),
        in_specs=[
            pl.BlockSpec((_BM, _D), lambda i: (i, 0)),
            pl.BlockSpec((_D, _D), lambda i: (0, 0)),
            pl.BlockSpec((1, _D), lambda i: (0, 0)),
        ],
        out_specs=pl.BlockSpec((_BM, _D), lambda i: (i, 0)),
        out_shape=jax.ShapeDtypeStruct((n, _D), jnp.float32),
    )(x, W, b.reshape(1, _D))


def _mm_out(x, pa, pb, Ws, WnA, WnB, bo):
    def body(x_ref, pa_ref, pb_ref, ws_ref, wna_ref, wnb_ref, b_ref, o_ref):
        acc = jnp.dot(x_ref[...], ws_ref[...],
                      preferred_element_type=jnp.float32)
        acc = acc + jnp.dot(pa_ref[...], wna_ref[...],
                            preferred_element_type=jnp.float32)
        acc = acc + jnp.dot(pb_ref[...], wnb_ref[...],
                            preferred_element_type=jnp.float32)
        o_ref[...] = jnp.maximum(acc + b_ref[...], 0.0)

    n = x.shape[0]
    return pl.pallas_call(
        body,
        grid=(n // _BM,),
        in_specs=[
            pl.BlockSpec((_BM, _D), lambda i: (i, 0)),
            pl.BlockSpec((_BM, _DH), lambda i: (i, 0)),
            pl.BlockSpec((_BM, _DH), lambda i: (i, 0)),
            pl.BlockSpec((_D, _D), lambda i: (0, 0)),
            pl.BlockSpec((_DH, _D), lambda i: (0, 0)),
            pl.BlockSpec((_DH, _D), lambda i: (0, 0)),
            pl.BlockSpec((1, _D), lambda i: (0, 0)),
        ],
        out_specs=pl.BlockSpec((_BM, _D), lambda i: (i, 0)),
        out_shape=jax.ShapeDtypeStruct((n, _D), jnp.float32),
    )(x, pa, pb, Ws, WnA, WnB, bo.reshape(1, _D))


def _loss_call(SA, SB, gt, W1, b1c, W2, b2c):
    Nr, Mr = gt.shape

    def body(sa_ref, sb_ref, gt_ref, w1_ref, b1_ref, w2_ref, b2_ref,
             pred_ref, aff_ref, trip_ref, bm_ref):
        SA_ = sa_ref[...]
        SB_ = sb_ref[...]
        gt_ = gt_ref[...]
        w1 = w1_ref[...]
        A1T = lax.dot_general(w1, SA_, (((0,), (1,)), ((), ())),
                              preferred_element_type=jnp.float32,
                              precision=lax.Precision.HIGHEST)
        A1T = A1T + b1_ref[...]
        bm_ref[...] = jnp.dot(SB_, w1, preferred_element_type=jnp.float32,
                              precision=lax.Precision.HIGHEST)
        w2c = w2_ref[...]
        b2s = b2_ref[0, 0]

        def iblk(ib, aff_sum):
            i0 = ib * 8
            Bblk = bm_ref[pl.ds(i0, 8), :][:, :, None]
            Z = A1T[None, :, :] - Bblk
            T = jnp.maximum(Z, 0.0) * w2c[None, :, :]
            P = jnp.sum(T, axis=1) + b2s
            pred_ref[pl.ds(i0, 8), :] = P
            g = gt_ref[pl.ds(i0, 8), :]
            return aff_sum + jnp.sum(
                jnp.maximum(P, 0.0) - P * g
                + jnp.log1p(jnp.exp(-jnp.abs(P))))

        aff_sum = lax.fori_loop(0, Nr // 8, iblk, jnp.float32(0.0))
        aff_ref[0, 0] = aff_sum / jnp.float32(Nr * Mr)

        ra = jnp.sum(SB_ * SB_, axis=1)[:, None]
        rb = jnp.sum(SA_ * SA_, axis=1)[None, :]
        G = lax.dot_general(SB_, SA_, (((1,), (1,)), ((), ())),
                            preferred_element_type=jnp.float32,
                            precision=lax.Precision.HIGHEST)
        d2 = ra + rb - 2.0 * G
        Dm = jnp.sqrt(jnp.maximum(d2, 1e-12))
        d_pos = jnp.max(Dm * gt_, axis=1)
        big = jnp.float32(3.0e38)
        d_neg = jnp.min(jnp.where(gt_ < 0.5, Dm, big), axis=1)
        d_neg = jnp.where(d_neg > jnp.float32(1.0e37), 0.0, d_neg)
        trip_ref[0, 0] = jnp.mean(jnp.maximum(d_pos - d_neg + 10.0, 0.0))

    return pl.pallas_call(
        body,
        in_specs=[
            pl.BlockSpec((Nr, _D), lambda: (0, 0)),
            pl.BlockSpec((Nr, _D), lambda: (0, 0)),
            pl.BlockSpec((Nr, Mr), lambda: (0, 0)),
            pl.BlockSpec((_D, _HID), lambda: (0, 0)),
            pl.BlockSpec((_HID, 1), lambda: (0, 0)),
            pl.BlockSpec((_HID, 1), lambda: (0, 0)),
            pl.BlockSpec((1, 1), lambda: (0, 0)),
        ],
        out_specs=[
            pl.BlockSpec((Nr, Mr), lambda: (0, 0)),
            pl.BlockSpec(memory_space=pltpu.SMEM),
            pl.BlockSpec(memory_space=pltpu.SMEM),
        ],
        out_shape=[
            jax.ShapeDtypeStruct((Nr, Mr), jnp.float32),
            jax.ShapeDtypeStruct((1, 1), jnp.float32),
            jax.ShapeDtypeStruct((1, 1), jnp.float32),
        ],
        scratch_shapes=[pltpu.VMEM((Nr, _HID), jnp.float32)],
    )(SA, SB, gt, W1, b1c, W2, b2c)


def kernel(embeddings, gt_aff_mat, edge_index, N, M, W_pool, b_pool, W_self,
           W_neigh, b_out, W1, b1, W2, b2):
    src = edge_index[0]
    dst = edge_index[1]
    Nr, Mr = gt_aff_mat.shape
    xp = jnp.zeros((_NPAD, _D), jnp.float32).at[:_N_NODES].set(embeddings)
    b1c = b1.reshape(_HID, 1)
    b2c = b2.reshape(1, 1)
    aff = jnp.float32(0.0)
    trip = jnp.float32(0.0)
    pred = None
    csrc, cdst, cnt = _edge_lists(src, dst)
    for l in range(_L):
        h = _mm_relu(xp, W_pool[l], b_pool[l])
        pa, pb = _segmax(h, csrc, cdst, cnt)
        xp = _mm_out(xp, pa, pb, W_self[l], W_neigh[l][:_DH],
                     W_neigh[l][_DH:], b_out[l])
        SA = lax.dynamic_slice(xp, (N, 0), (Mr, _D))
        SB = lax.dynamic_slice(xp, (M - Mr, 0), (Nr, _D))
        pred, aff_l, trip_l = _loss_call(SA, SB, gt_aff_mat, W1, b1c, W2, b2c)
        aff = aff + aff_l[0, 0]
        trip = trip + trip_l[0, 0]
    total = trip + aff
    return (total, trip, aff, pred)
